# Initial kernel scaffold; baseline (speedup 1.0000x reference)
import jax
import jax.numpy as jnp
from jax.experimental import pallas as pl


def kernel(user_emb, item_emb, user_intent, item_intent, all_h_list, all_t_list):
    raise NotImplementedError("write your pallas kernel here")


# speedup vs baseline: 5.8320x; 5.8320x over previous
import functools

import jax
import jax.numpy as jnp
from jax import lax
from jax.experimental import pallas as pl
from jax.experimental.pallas import tpu as pltpu
from jax.experimental.pallas import tpu_sc as plsc

N_USERS = 25000
N_ITEMS = 25000
N = N_USERS + N_ITEMS
E = 800000
D = 32
K_INT = 128
N_LAYERS = 2

NC = 2
NS = 16
NW = NC * NS

C = 128
NCH = E // C
NJ = (NCH + NW - 1) // NW
NPAD = 51200
ROWS_PER_TILE = NPAD // NS

_MESH = plsc.VectorSubcoreMesh(
    core_axis_name="c", subcore_axis_name="s", num_cores=NC, num_subcores=NS
)
_SC_PARAMS = pltpu.CompilerParams(
    use_tc_tiling_on_sc=False, needs_layout_passes=False
)


def _worker_ids():
  cid = lax.axis_index("c")
  sid = lax.axis_index("s")
  return cid, sid, sid * NC + cid


def _iota16():
  return lax.iota(jnp.int32, 16)


@functools.partial(
    pl.kernel,
    out_type=jax.ShapeDtypeStruct((NC, NPAD, 16), jnp.float32),
    mesh=_MESH,
    compiler_params=_SC_PARAMS,
    scratch_types=[
        pltpu.VMEM((C,), jnp.int32),
        pltpu.VMEM((C, 16), jnp.float32),
        pltpu.VMEM_SHARED((NPAD, 16), jnp.float32),
    ],
)
def _sc_deg(h_hbm, ones_hbm, zeros_hbm, out_hbm, hbuf, vbuf, acc):
  cid, sid, wid = _worker_ids()
  r0 = sid * ROWS_PER_TILE
  pltpu.sync_copy(zeros_hbm.at[pl.ds(0, ROWS_PER_TILE)],
                  acc.at[pl.ds(r0, ROWS_PER_TILE)])
  pltpu.sync_copy(ones_hbm, vbuf)
  plsc.subcore_barrier()

  def body(j, carry):
    c = j * NW + wid

    @pl.when(c < NCH)
    def _():
      pltpu.sync_copy(h_hbm.at[pl.ds(c * C, C)], hbuf)
      pltpu.sync_copy(vbuf, acc.at[hbuf], add=True)

    return carry

  lax.fori_loop(0, NJ, body, 0)
  plsc.subcore_barrier()
  pltpu.sync_copy(acc.at[pl.ds(r0, ROWS_PER_TILE)],
                  out_hbm.at[cid, pl.ds(r0, ROWS_PER_TILE)])


def _make_sc_spmm(scaled: bool):
  scratch = [
      pltpu.VMEM((C,), jnp.int32),
      pltpu.VMEM((C,), jnp.int32),
      pltpu.VMEM((C, D), jnp.float32),
      pltpu.VMEM_SHARED((NPAD, D), jnp.float32),
      pltpu.SemaphoreType.DMA,
  ]
  if scaled:
    scratch.append(pltpu.VMEM((C,), jnp.float32))

  def body_fn(table_hbm, h_hbm, t_hbm, alpha_hbm, zeros_hbm, out_hbm,
              hbuf, tbuf, gbuf, acc, sem, abuf=None):
    cid, sid, wid = _worker_ids()
    r0 = sid * ROWS_PER_TILE
    pltpu.sync_copy(zeros_hbm.at[pl.ds(0, ROWS_PER_TILE)],
                    acc.at[pl.ds(r0, ROWS_PER_TILE)])
    plsc.subcore_barrier()

    def body(j, carry):
      c = j * NW + wid

      @pl.when(c < NCH)
      def _():
        base = c * C
        pltpu.sync_copy(t_hbm.at[pl.ds(base, C)], tbuf)
        pltpu.sync_copy(h_hbm.at[pl.ds(base, C)], hbuf)
        pltpu.async_copy(table_hbm.at[tbuf], gbuf, sem).wait()
        if scaled:
          pltpu.sync_copy(alpha_hbm.at[pl.ds(base, C)], abuf)

          def scale_group(g, carry2):
            ridx = g * 16 + _iota16()
            av = abuf[pl.ds(g * 16, 16)]
            for d in range(D):
              dcol = jnp.full((16,), d, jnp.int32)
              v = plsc.load_gather(gbuf, [ridx, dcol])
              plsc.store_scatter(gbuf, [ridx, dcol], v * av)
            return carry2

          lax.fori_loop(0, C // 16, scale_group, 0)
        pltpu.sync_copy(gbuf, acc.at[hbuf], add=True)

      return carry

    lax.fori_loop(0, NJ, body, 0)
    plsc.subcore_barrier()
    pltpu.sync_copy(acc.at[pl.ds(r0, ROWS_PER_TILE)],
                    out_hbm.at[cid, pl.ds(r0, ROWS_PER_TILE)])

  if not scaled:
    def body_unscaled(table_hbm, h_hbm, t_hbm, zeros_hbm, out_hbm,
                      hbuf, tbuf, gbuf, acc, sem):
      body_fn(table_hbm, h_hbm, t_hbm, None, zeros_hbm, out_hbm,
              hbuf, tbuf, gbuf, acc, sem)
    fn = body_unscaled
  else:
    fn = body_fn

  return pl.kernel(
      fn,
      out_type=jax.ShapeDtypeStruct((NC, NPAD, D), jnp.float32),
      mesh=_MESH,
      compiler_params=_SC_PARAMS,
      scratch_types=scratch,
  )


_sc_spmm_plain = _make_sc_spmm(False)
_sc_spmm_scaled = _make_sc_spmm(True)


@functools.partial(
    pl.kernel,
    out_type=(
        jax.ShapeDtypeStruct((E,), jnp.float32),
        jax.ShapeDtypeStruct((E,), jnp.float32),
        jax.ShapeDtypeStruct((NC, NPAD, 16), jnp.float32),
    ),
    mesh=_MESH,
    compiler_params=_SC_PARAMS,
    scratch_types=[
        pltpu.VMEM((C,), jnp.int32),
        pltpu.VMEM((C,), jnp.int32),
        pltpu.VMEM((C, 2 * D), jnp.float32),
        pltpu.VMEM((C, 2 * D), jnp.float32),
        pltpu.VMEM((C, 16), jnp.float32),
        pltpu.VMEM((C,), jnp.float32),
        pltpu.VMEM((C,), jnp.float32),
        pltpu.VMEM_SHARED((NPAD, 16), jnp.float32),
        pltpu.SemaphoreType.DMA,
        pltpu.SemaphoreType.DMA,
    ],
)
def _sc_alpha(hgi_hbm, h_hbm, t_hbm, zeros_hbm, ag_hbm, ai_hbm, rs_hbm,
              hbuf, tbuf, ghb, gtb, vbuf, abg, abi, acc, sem1, sem2):
  cid, sid, wid = _worker_ids()
  r0 = sid * ROWS_PER_TILE
  pltpu.sync_copy(zeros_hbm.at[pl.ds(0, ROWS_PER_TILE)],
                  acc.at[pl.ds(r0, ROWS_PER_TILE)])
  pltpu.sync_copy(zeros_hbm.at[pl.ds(0, C)], vbuf)
  plsc.subcore_barrier()

  def body(j, carry):
    c = j * NW + wid

    @pl.when(c < NCH)
    def _():
      base = c * C
      pltpu.sync_copy(h_hbm.at[pl.ds(base, C)], hbuf)
      pltpu.sync_copy(t_hbm.at[pl.ds(base, C)], tbuf)
      cp1 = pltpu.async_copy(hgi_hbm.at[hbuf], ghb, sem1)
      cp2 = pltpu.async_copy(hgi_hbm.at[tbuf], gtb, sem2)
      cp1.wait()
      cp2.wait()

      def group(g, carry2):
        ridx = g * 16 + _iota16()
        acc_g = jnp.zeros((16,), jnp.float32)
        acc_i = jnp.zeros((16,), jnp.float32)
        for d in range(2 * D):
          dcol = jnp.full((16,), d, jnp.int32)
          v1 = plsc.load_gather(ghb, [ridx, dcol])
          v2 = plsc.load_gather(gtb, [ridx, dcol])
          if d < D:
            acc_g = acc_g + v1 * v2
          else:
            acc_i = acc_i + v1 * v2
        ag = acc_g * 0.5 + 0.5
        ai = acc_i * 0.5 + 0.5
        abg[pl.ds(g * 16, 16)] = ag
        abi[pl.ds(g * 16, 16)] = ai
        plsc.store_scatter(vbuf, [ridx, jnp.zeros((16,), jnp.int32)], ag)
        plsc.store_scatter(vbuf, [ridx, jnp.ones((16,), jnp.int32)], ai)
        return carry2

      lax.fori_loop(0, C // 16, group, 0)
      pltpu.sync_copy(abg, ag_hbm.at[pl.ds(base, C)])
      pltpu.sync_copy(abi, ai_hbm.at[pl.ds(base, C)])
      pltpu.sync_copy(vbuf, acc.at[hbuf], add=True)

    return carry

  lax.fori_loop(0, NJ, body, 0)
  plsc.subcore_barrier()
  pltpu.sync_copy(acc.at[pl.ds(r0, ROWS_PER_TILE)],
                  rs_hbm.at[cid, pl.ds(r0, ROWS_PER_TILE)])


_BLK = 1000
_NBLK = N // _BLK


def _tc0_body(degacc, cur, dis32, y1):
  deg = degacc[0, :, 0] + degacc[1, :, 0]
  dis = jnp.where(deg > 0, lax.rsqrt(jnp.maximum(deg, 1e-30)), 0.0)
  d32 = jnp.broadcast_to(dis[:, None], (_BLK, D))
  dis32[...] = d32
  y1[...] = d32 * cur[...]


def _tc0(degacc, cur0):
  return pl.pallas_call(
      _tc0_body,
      grid=(_NBLK,),
      in_specs=[
          pl.BlockSpec((NC, _BLK, 16), lambda b: (0, b, 0)),
          pl.BlockSpec((_BLK, D), lambda b: (b, 0)),
      ],
      out_specs=[
          pl.BlockSpec((_BLK, D), lambda b: (b, 0)),
          pl.BlockSpec((_BLK, D), lambda b: (b, 0)),
      ],
      out_shape=[
          jax.ShapeDtypeStruct((N, D), jnp.float32),
          jax.ShapeDtypeStruct((N, D), jnp.float32),
      ],
  )(degacc, cur0)


def _l2n(x):
  nrm = jnp.sqrt(jnp.sum(x * x, axis=1, keepdims=True))
  return x / jnp.maximum(nrm, 1e-8)


def _tc1_body(gnnacc, cur, dis32, ui, ii, gnn, intl, hgi):
  s = gnnacc[0] + gnnacc[1]
  g = dis32[...] * s
  gnn[...] = g
  b = pl.program_id(0)
  w = jnp.where(b < _NBLK // 2, ui[...], ii[...])
  logits = jnp.dot(cur[...], w, preferred_element_type=jnp.float32)
  m = jnp.max(logits, axis=1, keepdims=True)
  e = jnp.exp(logits - m)
  p = e / jnp.sum(e, axis=1, keepdims=True)
  il = jnp.dot(p, w.T, preferred_element_type=jnp.float32)
  intl[...] = il
  hgi[...] = jnp.concatenate([_l2n(g), _l2n(il)], axis=1)


def _tc1(gnnacc, cur, dis32, ui, ii):
  return pl.pallas_call(
      _tc1_body,
      grid=(_NBLK,),
      in_specs=[
          pl.BlockSpec((NC, _BLK, D), lambda b: (0, b, 0)),
          pl.BlockSpec((_BLK, D), lambda b: (b, 0)),
          pl.BlockSpec((_BLK, D), lambda b: (b, 0)),
          pl.BlockSpec((D, K_INT), lambda b: (0, 0)),
          pl.BlockSpec((D, K_INT), lambda b: (0, 0)),
      ],
      out_specs=[
          pl.BlockSpec((_BLK, D), lambda b: (b, 0)),
          pl.BlockSpec((_BLK, D), lambda b: (b, 0)),
          pl.BlockSpec((_BLK, 2 * D), lambda b: (b, 0)),
      ],
      out_shape=[
          jax.ShapeDtypeStruct((N, D), jnp.float32),
          jax.ShapeDtypeStruct((N, D), jnp.float32),
          jax.ShapeDtypeStruct((N, 2 * D), jnp.float32),
      ],
  )(gnnacc, cur, dis32, ui, ii)


def _tc2_body(gaacc, iaacc, rsacc, gnn, intl, cur, dis32,
              gaa, iaa, nxt, ynxt):
  rs = rsacc[0] + rsacc[1]
  dg = rs[:, 0:1]
  di = rs[:, 1:2]
  inv_g = jnp.where(dg != 0, 1.0 / jnp.where(dg != 0, dg, 1.0), 0.0)
  inv_i = jnp.where(di != 0, 1.0 / jnp.where(di != 0, di, 1.0), 0.0)
  ga = inv_g * (gaacc[0] + gaacc[1])
  ia = inv_i * (iaacc[0] + iaacc[1])
  gaa[...] = ga
  iaa[...] = ia
  nx = gnn[...] + intl[...] + ga + ia + cur[...]
  nxt[...] = nx
  ynxt[...] = dis32[...] * nx


def _tc2(gaacc, iaacc, rsacc, gnn, intl, cur, dis32):
  return pl.pallas_call(
      _tc2_body,
      grid=(_NBLK,),
      in_specs=[
          pl.BlockSpec((NC, _BLK, D), lambda b: (0, b, 0)),
          pl.BlockSpec((NC, _BLK, D), lambda b: (0, b, 0)),
          pl.BlockSpec((NC, _BLK, 16), lambda b: (0, b, 0)),
          pl.BlockSpec((_BLK, D), lambda b: (b, 0)),
          pl.BlockSpec((_BLK, D), lambda b: (b, 0)),
          pl.BlockSpec((_BLK, D), lambda b: (b, 0)),
          pl.BlockSpec((_BLK, D), lambda b: (b, 0)),
      ],
      out_specs=[pl.BlockSpec((_BLK, D), lambda b: (b, 0))] * 4,
      out_shape=[jax.ShapeDtypeStruct((N, D), jnp.float32)] * 4,
  )(gaacc, iaacc, rsacc, gnn, intl, cur, dis32)


def kernel(user_emb, item_emb, user_intent, item_intent, all_h_list,
           all_t_list):
  h = all_h_list.astype(jnp.int32)
  t = all_t_list.astype(jnp.int32)
  cur = jnp.concatenate([user_emb, item_emb], axis=0)

  zeros32 = jnp.zeros((ROWS_PER_TILE, D), jnp.float32)
  zeros16 = jnp.zeros((ROWS_PER_TILE, 16), jnp.float32)
  ones16 = jnp.zeros((C, 16), jnp.float32).at[:, 0].set(1.0)

  degacc = _sc_deg(h, ones16, zeros16)
  dis32, y = _tc0(degacc, cur)

  gnns, ints, gaas, iaas = [], [], [], []
  for _ in range(N_LAYERS):
    gnnacc = _sc_spmm_plain(y, h, t, zeros32)
    gnn, intl, hgi = _tc1(gnnacc, cur, dis32, user_intent, item_intent)
    ag, ai, rsacc = _sc_alpha(hgi, h, t, zeros16)
    gaacc = _sc_spmm_scaled(cur, h, t, ag, zeros32)
    iaacc = _sc_spmm_scaled(cur, h, t, ai, zeros32)
    gaa, iaa, nxt, ynxt = _tc2(gaacc, iaacc, rsacc, gnn, intl, cur, dis32)
    gnns.append(gnn)
    ints.append(intl)
    gaas.append(gaa)
    iaas.append(iaa)
    cur = nxt
    y = ynxt

  return jnp.stack(gnns + ints + gaas + iaas, axis=0)


# speedup vs baseline: 6.8287x; 1.1709x over previous
import functools

import jax
import jax.numpy as jnp
from jax import lax
from jax.experimental import pallas as pl
from jax.experimental.pallas import tpu as pltpu
from jax.experimental.pallas import tpu_sc as plsc

N_USERS = 25000
N_ITEMS = 25000
N = N_USERS + N_ITEMS
E = 800000
D = 32
K_INT = 128
N_LAYERS = 2

NC = 2
NS = 16
NW = NC * NS

C = 128
NCH = 6272
EP = NCH * C
NPAD = 51200
ROWS_PER_TILE = NPAD // NS
TRASH = NPAD - 1

_MESH = plsc.VectorSubcoreMesh(
    core_axis_name="c", subcore_axis_name="s", num_cores=NC, num_subcores=NS
)
_SC_PARAMS = pltpu.CompilerParams(
    use_tc_tiling_on_sc=False, needs_layout_passes=False,
    has_side_effects=True
)


def _worker_ids():
  cid = lax.axis_index("c")
  sid = lax.axis_index("s")
  return cid, sid, sid * NC + cid


def _iota16():
  return lax.iota(jnp.int32, 16)


KD = 14
NJD = NCH // (KD * NW)


@functools.partial(
    pl.kernel,
    out_type=jax.ShapeDtypeStruct((NC, NPAD, 16), jnp.float32),
    mesh=_MESH,
    compiler_params=_SC_PARAMS,
    scratch_types=[
        pltpu.VMEM((KD, C), jnp.int32),
        pltpu.VMEM((C, 16), jnp.float32),
        pltpu.VMEM_SHARED((NPAD, 16), jnp.float32),
        pltpu.SemaphoreType.DMA,
    ],
)
def _sc_deg(h2_hbm, ones_hbm, zeros_hbm, out_hbm, hbuf, vbuf, acc, ssem):
  cid, sid, wid = _worker_ids()
  r0 = sid * ROWS_PER_TILE
  pltpu.sync_copy(zeros_hbm.at[pl.ds(0, ROWS_PER_TILE)],
                  acc.at[pl.ds(r0, ROWS_PER_TILE)])
  pltpu.sync_copy(ones_hbm, vbuf)
  plsc.subcore_barrier()

  def body(j, carry):
    o = j * NW + wid
    pltpu.sync_copy(h2_hbm.at[pl.ds(o * KD, KD)], hbuf)
    for k in range(KD):
      pltpu.sync_copy(vbuf, acc.at[hbuf.at[k]], add=True)
    return carry

  lax.fori_loop(0, NJD, body, 0)
  plsc.subcore_barrier()
  pltpu.sync_copy(acc.at[pl.ds(r0, ROWS_PER_TILE)],
                  out_hbm.at[cid, pl.ds(r0, ROWS_PER_TILE)])


KS = 4
NJS = NCH // (KS * NW)


def _make_sc_spmm(scaled: bool):
  scratch = [
      pltpu.VMEM((KS, C), jnp.int32),
      pltpu.VMEM((KS, C), jnp.int32),
      pltpu.VMEM((KS * C, D), jnp.float32),
      pltpu.VMEM_SHARED((NPAD, D), jnp.float32),
      pltpu.SemaphoreType.DMA,
  ] + [pltpu.SemaphoreType.DMA for _ in range(KS)]
  if scaled:
    scratch.append(pltpu.VMEM((KS, C), jnp.float32))

  def body_fn(table_hbm, h2_hbm, t2_hbm, a2_hbm, zeros_hbm, out_hbm,
              hbuf, tbuf, gball, acc, ssem, *rest):
    gsems = list(rest[:KS])
    abuf = rest[KS] if scaled else None
    cid, sid, wid = _worker_ids()
    r0 = sid * ROWS_PER_TILE
    pltpu.sync_copy(zeros_hbm.at[pl.ds(0, ROWS_PER_TILE)],
                    acc.at[pl.ds(r0, ROWS_PER_TILE)])
    plsc.subcore_barrier()

    def body(j, carry):
      o = j * NW + wid
      pltpu.sync_copy(t2_hbm.at[pl.ds(o * KS, KS)], tbuf)
      pltpu.sync_copy(h2_hbm.at[pl.ds(o * KS, KS)], hbuf)
      if scaled:
        pltpu.sync_copy(a2_hbm.at[pl.ds(o * KS, KS)], abuf)
      gcps = [pltpu.async_copy(table_hbm.at[tbuf.at[k]],
                               gball.at[pl.ds(k * C, C)], gsems[k])
              for k in range(KS)]
      for k in range(KS):
        gcps[k].wait()
        if scaled:
          def scale_group(g, carry2, k=k):
            ridx = k * C + g * 16 + _iota16()
            av = abuf[k, pl.ds(g * 16, 16)]
            for d in range(D):
              dcol = jnp.full((16,), d, jnp.int32)
              v = plsc.load_gather(gball, [ridx, dcol])
              plsc.store_scatter(gball, [ridx, dcol], v * av)
            return carry2

          lax.fori_loop(0, C // 16, scale_group, 0)
        pltpu.sync_copy(gball.at[pl.ds(k * C, C)], acc.at[hbuf.at[k]],
                        add=True)
      return carry

    lax.fori_loop(0, NJS, body, 0)
    plsc.subcore_barrier()
    pltpu.sync_copy(acc.at[pl.ds(r0, ROWS_PER_TILE)],
                    out_hbm.at[cid, pl.ds(r0, ROWS_PER_TILE)])

  if not scaled:
    def body_unscaled(table_hbm, h2_hbm, t2_hbm, zeros_hbm, out_hbm,
                      hbuf, tbuf, gball, acc, ssem, *rest):
      body_fn(table_hbm, h2_hbm, t2_hbm, None, zeros_hbm, out_hbm,
              hbuf, tbuf, gball, acc, ssem, *rest)
    fn = body_unscaled
  else:
    fn = body_fn

  return pl.kernel(
      fn,
      out_type=jax.ShapeDtypeStruct((NC, NPAD, D), jnp.float32),
      mesh=_MESH,
      compiler_params=_SC_PARAMS,
      scratch_types=scratch,
  )


_sc_spmm_plain = _make_sc_spmm(False)
_sc_spmm_scaled = _make_sc_spmm(True)


K2 = 2
NJ2 = NCH // (K2 * NW)


@functools.partial(
    pl.kernel,
    out_type=(
        jax.ShapeDtypeStruct((NCH, C), jnp.float32),
        jax.ShapeDtypeStruct((NCH, C), jnp.float32),
        jax.ShapeDtypeStruct((NC, NPAD, 16), jnp.float32),
    ),
    mesh=_MESH,
    compiler_params=_SC_PARAMS,
    scratch_types=[
        pltpu.VMEM((K2, C), jnp.int32),
        pltpu.VMEM((K2, C), jnp.int32),
        pltpu.VMEM((K2 * C, 2 * D), jnp.float32),
        pltpu.VMEM((K2 * C, 2 * D), jnp.float32),
        pltpu.VMEM((K2 * C, 16), jnp.float32),
        pltpu.VMEM((K2, C), jnp.float32),
        pltpu.VMEM((K2, C), jnp.float32),
        pltpu.VMEM_SHARED((NPAD, 16), jnp.float32),
        pltpu.SemaphoreType.DMA,
        pltpu.SemaphoreType.DMA,
        pltpu.SemaphoreType.DMA,
        pltpu.SemaphoreType.DMA,
        pltpu.SemaphoreType.DMA,
    ],
)
def _sc_alpha(hgi_hbm, h2_hbm, t2_hbm, zeros_hbm, ag_hbm, ai_hbm, rs_hbm,
              hbuf, tbuf, ghall, gtall, vball, abg, abi, acc,
              gs0, gs1, gs2, gs3, ssem):
  gsems = [[gs0, gs1], [gs2, gs3]]
  cid, sid, wid = _worker_ids()
  r0 = sid * ROWS_PER_TILE
  pltpu.sync_copy(zeros_hbm.at[pl.ds(0, ROWS_PER_TILE)],
                  acc.at[pl.ds(r0, ROWS_PER_TILE)])
  pltpu.sync_copy(zeros_hbm.at[pl.ds(0, K2 * C)], vball)
  plsc.subcore_barrier()

  def body(j, carry):
    o = j * NW + wid
    pltpu.sync_copy(h2_hbm.at[pl.ds(o * K2, K2)], hbuf)
    pltpu.sync_copy(t2_hbm.at[pl.ds(o * K2, K2)], tbuf)
    gcps = [(pltpu.async_copy(hgi_hbm.at[hbuf.at[k]],
                              ghall.at[pl.ds(k * C, C)], gsems[k][0]),
             pltpu.async_copy(hgi_hbm.at[tbuf.at[k]],
                              gtall.at[pl.ds(k * C, C)], gsems[k][1]))
            for k in range(K2)]
    for k in range(K2):
      gcps[k][0].wait()
      gcps[k][1].wait()

      def group(g, carry2, k=k):
        ridx = k * C + g * 16 + _iota16()
        acc_g = jnp.zeros((16,), jnp.float32)
        acc_i = jnp.zeros((16,), jnp.float32)
        for d in range(2 * D):
          dcol = jnp.full((16,), d, jnp.int32)
          v1 = plsc.load_gather(ghall, [ridx, dcol])
          v2 = plsc.load_gather(gtall, [ridx, dcol])
          if d < D:
            acc_g = acc_g + v1 * v2
          else:
            acc_i = acc_i + v1 * v2
        ag = acc_g * 0.5 + 0.5
        ai = acc_i * 0.5 + 0.5
        abg[k, pl.ds(g * 16, 16)] = ag
        abi[k, pl.ds(g * 16, 16)] = ai
        plsc.store_scatter(vball, [ridx, jnp.zeros((16,), jnp.int32)], ag)
        plsc.store_scatter(vball, [ridx, jnp.ones((16,), jnp.int32)], ai)
        return carry2

      lax.fori_loop(0, C // 16, group, 0)
      pltpu.sync_copy(vball.at[pl.ds(k * C, C)], acc.at[hbuf.at[k]],
                      add=True)
      pltpu.sync_copy(abg.at[k], ag_hbm.at[o * K2 + k])
      pltpu.sync_copy(abi.at[k], ai_hbm.at[o * K2 + k])
    return carry

  lax.fori_loop(0, NJ2, body, 0)
  plsc.subcore_barrier()
  pltpu.sync_copy(acc.at[pl.ds(r0, ROWS_PER_TILE)],
                  rs_hbm.at[cid, pl.ds(r0, ROWS_PER_TILE)])


_BLK = 1000
_NBLK = N // _BLK


def _tc0_body(degacc, cur, dis32, y1):
  deg = degacc[0, :, 0] + degacc[1, :, 0]
  dis = jnp.where(deg > 0, lax.rsqrt(jnp.maximum(deg, 1e-30)), 0.0)
  d32 = jnp.broadcast_to(dis[:, None], (_BLK, D))
  dis32[...] = d32
  y1[...] = d32 * cur[...]


def _tc0(degacc, cur0):
  return pl.pallas_call(
      _tc0_body,
      grid=(_NBLK,),
      in_specs=[
          pl.BlockSpec((NC, _BLK, 16), lambda b: (0, b, 0)),
          pl.BlockSpec((_BLK, D), lambda b: (b, 0)),
      ],
      out_specs=[
          pl.BlockSpec((_BLK, D), lambda b: (b, 0)),
          pl.BlockSpec((_BLK, D), lambda b: (b, 0)),
      ],
      out_shape=[
          jax.ShapeDtypeStruct((N, D), jnp.float32),
          jax.ShapeDtypeStruct((N, D), jnp.float32),
      ],
  )(degacc, cur0)


def _l2n(x):
  nrm = jnp.sqrt(jnp.sum(x * x, axis=1, keepdims=True))
  return x / jnp.maximum(nrm, 1e-8)


def _tc1_body(gnnacc, cur, dis32, ui, ii, gnn, intl, hgi):
  s = gnnacc[0] + gnnacc[1]
  g = dis32[...] * s
  gnn[...] = g
  b = pl.program_id(0)
  w = jnp.where(b < _NBLK // 2, ui[...], ii[...])
  logits = jnp.dot(cur[...], w, preferred_element_type=jnp.float32)
  m = jnp.max(logits, axis=1, keepdims=True)
  e = jnp.exp(logits - m)
  p = e / jnp.sum(e, axis=1, keepdims=True)
  il = jnp.dot(p, w.T, preferred_element_type=jnp.float32)
  intl[...] = il
  hgi[...] = jnp.concatenate([_l2n(g), _l2n(il)], axis=1)


def _tc1(gnnacc, cur, dis32, ui, ii):
  return pl.pallas_call(
      _tc1_body,
      grid=(_NBLK,),
      in_specs=[
          pl.BlockSpec((NC, _BLK, D), lambda b: (0, b, 0)),
          pl.BlockSpec((_BLK, D), lambda b: (b, 0)),
          pl.BlockSpec((_BLK, D), lambda b: (b, 0)),
          pl.BlockSpec((D, K_INT), lambda b: (0, 0)),
          pl.BlockSpec((D, K_INT), lambda b: (0, 0)),
      ],
      out_specs=[
          pl.BlockSpec((_BLK, D), lambda b: (b, 0)),
          pl.BlockSpec((_BLK, D), lambda b: (b, 0)),
          pl.BlockSpec((_BLK, 2 * D), lambda b: (b, 0)),
      ],
      out_shape=[
          jax.ShapeDtypeStruct((N, D), jnp.float32),
          jax.ShapeDtypeStruct((N, D), jnp.float32),
          jax.ShapeDtypeStruct((N, 2 * D), jnp.float32),
      ],
  )(gnnacc, cur, dis32, ui, ii)


def _tc2_body(gaacc, iaacc, rsacc, gnn, intl, cur, dis32,
              gaa, iaa, nxt, ynxt):
  rs = rsacc[0] + rsacc[1]
  dg = rs[:, 0:1]
  di = rs[:, 1:2]
  inv_g = jnp.where(dg != 0, 1.0 / jnp.where(dg != 0, dg, 1.0), 0.0)
  inv_i = jnp.where(di != 0, 1.0 / jnp.where(di != 0, di, 1.0), 0.0)
  ga = inv_g * (gaacc[0] + gaacc[1])
  ia = inv_i * (iaacc[0] + iaacc[1])
  gaa[...] = ga
  iaa[...] = ia
  nx = gnn[...] + intl[...] + ga + ia + cur[...]
  nxt[...] = nx
  ynxt[...] = dis32[...] * nx


def _tc2(gaacc, iaacc, rsacc, gnn, intl, cur, dis32):
  return pl.pallas_call(
      _tc2_body,
      grid=(_NBLK,),
      in_specs=[
          pl.BlockSpec((NC, _BLK, D), lambda b: (0, b, 0)),
          pl.BlockSpec((NC, _BLK, D), lambda b: (0, b, 0)),
          pl.BlockSpec((NC, _BLK, 16), lambda b: (0, b, 0)),
          pl.BlockSpec((_BLK, D), lambda b: (b, 0)),
          pl.BlockSpec((_BLK, D), lambda b: (b, 0)),
          pl.BlockSpec((_BLK, D), lambda b: (b, 0)),
          pl.BlockSpec((_BLK, D), lambda b: (b, 0)),
      ],
      out_specs=[pl.BlockSpec((_BLK, D), lambda b: (b, 0))] * 4,
      out_shape=[jax.ShapeDtypeStruct((N, D), jnp.float32)] * 4,
  )(gaacc, iaacc, rsacc, gnn, intl, cur, dis32)


def _padrows(x):
  return jnp.concatenate(
      [x, jnp.zeros((NPAD - N,) + x.shape[1:], x.dtype)], axis=0)


def kernel(user_emb, item_emb, user_intent, item_intent, all_h_list,
           all_t_list):
  npad_e = EP - E
  h = jnp.concatenate(
      [all_h_list.astype(jnp.int32),
       jnp.full((npad_e,), TRASH, jnp.int32)]).reshape(NCH, C)
  t = jnp.concatenate(
      [all_t_list.astype(jnp.int32),
       jnp.zeros((npad_e,), jnp.int32)]).reshape(NCH, C)
  cur = jnp.concatenate([user_emb, item_emb], axis=0)

  zeros32 = jnp.zeros((ROWS_PER_TILE, D), jnp.float32)
  zeros16 = jnp.zeros((ROWS_PER_TILE, 16), jnp.float32)
  ones16 = jnp.zeros((C, 16), jnp.float32).at[:, 0].set(1.0)

  degacc = _sc_deg(h, ones16, zeros16)
  dis32, y = _tc0(degacc, cur)

  gnns, ints, gaas, iaas = [], [], [], []
  for _ in range(N_LAYERS):
    gnnacc = _sc_spmm_plain(_padrows(y), h, t, zeros32)
    gnn, intl, hgi = _tc1(gnnacc, cur, dis32, user_intent, item_intent)
    ag, ai, rsacc = _sc_alpha(_padrows(hgi), h, t, zeros16)
    curp = _padrows(cur)
    gaacc = _sc_spmm_scaled(curp, h, t, ag, zeros32)
    iaacc = _sc_spmm_scaled(curp, h, t, ai, zeros32)
    gaa, iaa, nxt, ynxt = _tc2(gaacc, iaacc, rsacc, gnn, intl, cur, dis32)
    gnns.append(gnn)
    ints.append(intl)
    gaas.append(gaa)
    iaas.append(iaa)
    cur = nxt
    y = ynxt

  return jnp.stack(gnns + ints + gaas + iaas, axis=0)
